# baseline (speedup 1.0000x reference)
import functools

import jax
import jax.numpy as jnp
from jax import lax
from jax.experimental import pallas as pl
from jax.experimental.pallas import tpu as pltpu
from jax.experimental.pallas import tpu_sc as plsc

_NUM_BUCKETS = 32
_LANES = 16


def _make_sc_kernel(H, N, NC, NW, G, per_w):
    n_groups = per_w // G
    mesh = plsc.VectorSubcoreMesh(core_axis_name="c", subcore_axis_name="s")

    @functools.partial(
        pl.kernel,
        out_type=jax.ShapeDtypeStruct((H, N), jnp.float32),
        mesh=mesh,
        compiler_params=pltpu.CompilerParams(needs_layout_passes=False),
        scratch_types=[
            pltpu.VMEM((H * _NUM_BUCKETS,), jnp.float32),
            pltpu.VMEM((G,), jnp.int32),
            pltpu.VMEM((G,), jnp.float32),
            pltpu.VMEM((G,), jnp.float32),
            pltpu.VMEM((G,), jnp.float32),
            pltpu.VMEM((G,), jnp.float32),
            pltpu.SemaphoreType.DMA,
            pltpu.SemaphoreType.DMA,
            pltpu.SemaphoreType.DMA,
            pltpu.SemaphoreType.DMA,
        ],
    )
    def sc_kernel(tab_hbm, rb_hbm, qk_hbm, out_hbm,
                  tab_v, rb_v, in0, in1, o0, o1, si0, si1, so0, so1):
        wid = lax.axis_index("s") * NC + lax.axis_index("c")
        pltpu.sync_copy(tab_hbm, tab_v)
        ins = [in0, in1]
        outs = [o0, o1]
        isems = [si0, si1]
        osems = [so0, so1]
        base_w = wid * per_w
        for g in range(n_groups):
            base = base_w + g * G
            pltpu.sync_copy(rb_hbm.at[pl.ds(base, G)], rb_v)
            in_copies = [None, None]
            out_copies = [None, None]
            in_copies[0] = pltpu.async_copy(
                qk_hbm.at[0, pl.ds(base, G)], ins[0], isems[0])
            for h in range(H):
                s = h % 2
                in_copies[s].wait()
                if h + 1 < H:
                    ns = (h + 1) % 2
                    in_copies[ns] = pltpu.async_copy(
                        qk_hbm.at[h + 1, pl.ds(base, G)], ins[ns], isems[ns])
                if out_copies[s] is not None:
                    out_copies[s].wait()
                hoff = jnp.full((_LANES,), h * _NUM_BUCKETS, jnp.int32)
                in_s = ins[s]
                out_s = outs[s]

                @plsc.parallel_loop(0, G // _LANES, 1, unroll=8)
                def body(v):
                    off = v * _LANES
                    idx = rb_v[pl.ds(off, _LANES)] + hoff
                    gval = plsc.load_gather(tab_v, [idx])
                    out_s[pl.ds(off, _LANES)] = in_s[pl.ds(off, _LANES)] + gval

                out_copies[s] = pltpu.async_copy(
                    out_s, out_hbm.at[h, pl.ds(base, G)], osems[s])
            out_copies[0].wait()
            out_copies[1].wait()

    return sc_kernel


def kernel(qk_dots, rp_buckets, rel_bias_table):
    B, H, I, J = qk_dots.shape
    N = I * J
    qk_f = qk_dots.reshape(H, N)
    rb_f = rp_buckets.reshape(N)
    tab = (rel_bias_table + 1.0).T.reshape(H * _NUM_BUCKETS)

    info = plsc.get_sparse_core_info()
    NC, NS = info.num_cores, info.num_subcores
    NW = NC * NS
    per_w = N // NW
    G = 16384

    sc_kernel = _make_sc_kernel(H, N, NC, NW, G, per_w)
    out = sc_kernel(tab, rb_f, qk_f)
    return out.reshape(B, H, I, J)
